# baseline (speedup 1.0000x reference)
import functools

import jax
import jax.numpy as jnp
from jax import lax
from jax.experimental import pallas as pl
from jax.experimental.pallas import tpu as pltpu
from jax.experimental.pallas import tpu_sc as plsc


def _make_gather_sum(V, D, B, C):
    info = plsc.get_sparse_core_info()
    NC, NS, L = info.num_cores, info.num_subcores, info.num_lanes
    NW = NC * NS
    assert B % NW == 0 and (B // NW) % 8 == 0
    assert D % L == 0
    b_per_w = B // NW
    slices_per_row = D // L
    mesh = plsc.VectorSubcoreMesh(core_axis_name="c", subcore_axis_name="s")

    @functools.partial(
        pl.kernel,
        mesh=mesh,
        out_type=jax.ShapeDtypeStruct((B, D), jnp.float32),
        scratch_types=[
            pltpu.VMEM((C, b_per_w), jnp.int32),
            pltpu.VMEM((b_per_w, D), jnp.float32),
            pltpu.VMEM((b_per_w, D), jnp.float32),
            pltpu.VMEM((b_per_w, D), jnp.float32),
            pltpu.SemaphoreType.DMA,
            pltpu.SemaphoreType.DMA,
            pltpu.SemaphoreType.DMA,
        ],
        compiler_params=pltpu.CompilerParams(use_tc_tiling_on_sc=False),
    )
    def gather_sum(ctx_hbm, table_hbm, out_hbm, idx_v, acc_v, buf0, buf1,
                   sem0, sem1, sem_a):
        wid = lax.axis_index("s") * NC + lax.axis_index("c")
        base = wid * b_per_w
        pltpu.sync_copy(ctx_hbm.at[wid], idx_v)

        bufs = (buf0, buf1)
        sems = (sem0, sem1)

        def add_into_acc(src):
            def row_body(i, _):
                for jj in range(slices_per_row):
                    sl = pl.ds(jj * L, L)
                    acc_v[i, sl] = acc_v[i, sl] + src[i, sl]
                return 0
            lax.fori_loop(0, b_per_w, row_body, 0)

        c0 = pltpu.async_copy(table_hbm.at[idx_v.at[0]], acc_v, sem_a)
        copies = [c0]
        for j in range(1, C):
            nb = bufs[(j - 1) % 2]
            copies.append(
                pltpu.async_copy(table_hbm.at[idx_v.at[j]], nb,
                                 sems[(j - 1) % 2]))
            copies[j - 1].wait()
            if j >= 2:
                add_into_acc(bufs[j % 2])
        copies[C - 1].wait()
        if C >= 2:
            add_into_acc(bufs[C % 2])
        pltpu.sync_copy(acc_v, out_hbm.at[pl.ds(base, b_per_w)])

    return gather_sum


_BN = 512


def _mm_body(ctx_ref, w_ref, b_ref, out_ref):
    out_ref[...] = (
        jnp.dot(ctx_ref[...], w_ref[...],
                preferred_element_type=jnp.float32)
        + b_ref[...]
    )


def _matmul_bias(ctx_sum, dense_w, dense_b2d):
    B, D = ctx_sum.shape
    N = dense_w.shape[1]
    nblocks = pl.cdiv(N, _BN)
    return pl.pallas_call(
        _mm_body,
        grid=(nblocks,),
        in_specs=[
            pl.BlockSpec((B, D), lambda n: (0, 0)),
            pl.BlockSpec((D, _BN), lambda n: (0, n)),
            pl.BlockSpec((1, _BN), lambda n: (0, n)),
        ],
        out_specs=pl.BlockSpec((B, _BN), lambda n: (0, n)),
        out_shape=jax.ShapeDtypeStruct((B, N), jnp.float32),
    )(ctx_sum, dense_w, dense_b2d)


def kernel(context, emb_table, dense_w, dense_b):
    B, C = context.shape
    V, D = emb_table.shape
    N = dense_w.shape[1]
    info = plsc.get_sparse_core_info()
    NW = info.num_cores * info.num_subcores
    b_per_w = B // NW
    ctx_r = (context.astype(jnp.int32)
             .T.reshape(C, NW, b_per_w).transpose(1, 0, 2))
    gather_sum = _make_gather_sum(V, D, B, C)
    ctx_sum = gather_sum(ctx_r, emb_table)
    return _matmul_bias(ctx_sum, dense_w, dense_b.reshape(1, N))


# speedup vs baseline: 1.0033x; 1.0033x over previous
import functools

import jax
import jax.numpy as jnp
from jax import lax
from jax.experimental import pallas as pl
from jax.experimental.pallas import tpu as pltpu
from jax.experimental.pallas import tpu_sc as plsc

_DP = 128


def _make_gather_sum(V, D, B, C):
    info = plsc.get_sparse_core_info()
    NC, NS, L = info.num_cores, info.num_subcores, info.num_lanes
    NW = NC * NS
    assert B % NW == 0 and (B // NW) % 8 == 0
    assert D % L == 0
    b_per_w = B // NW
    slices_per_row = D // L
    mesh = plsc.VectorSubcoreMesh(core_axis_name="c", subcore_axis_name="s")

    @functools.partial(
        pl.kernel,
        mesh=mesh,
        out_type=jax.ShapeDtypeStruct((B, _DP), jnp.float32),
        scratch_types=[
            pltpu.VMEM((C, b_per_w), jnp.int32),
            pltpu.VMEM((b_per_w, _DP), jnp.float32),
            pltpu.VMEM((b_per_w, _DP), jnp.float32),
            pltpu.VMEM((b_per_w, _DP), jnp.float32),
            pltpu.SemaphoreType.DMA,
            pltpu.SemaphoreType.DMA,
            pltpu.SemaphoreType.DMA,
        ],
    )
    def gather_sum(ctx_hbm, table_hbm, out_hbm, idx_v, acc_v, buf0, buf1,
                   sem0, sem1, sem_a):
        wid = lax.axis_index("s") * NC + lax.axis_index("c")
        base = wid * b_per_w
        pltpu.sync_copy(ctx_hbm.at[wid], idx_v)

        bufs = (buf0, buf1)
        sems = (sem0, sem1)

        def add_into_acc(src):
            def row_body(i, _):
                for jj in range(slices_per_row):
                    sl = pl.ds(jj * L, L)
                    acc_v[i, sl] = acc_v[i, sl] + src[i, sl]
                return 0
            lax.fori_loop(0, b_per_w, row_body, 0)

        c0 = pltpu.async_copy(table_hbm.at[idx_v.at[0]], acc_v, sem_a)
        copies = [c0]
        for j in range(1, C):
            nb = bufs[(j - 1) % 2]
            copies.append(
                pltpu.async_copy(table_hbm.at[idx_v.at[j]], nb,
                                 sems[(j - 1) % 2]))
            copies[j - 1].wait()
            if j >= 2:
                add_into_acc(bufs[j % 2])
        copies[C - 1].wait()
        if C >= 2:
            add_into_acc(bufs[C % 2])
        pltpu.sync_copy(acc_v, out_hbm.at[pl.ds(base, b_per_w)])

    return gather_sum


_BN = 512


def _mm_body(ctx_ref, w_ref, b_ref, out_ref):
    out_ref[...] = (
        jnp.dot(ctx_ref[...], w_ref[...],
                preferred_element_type=jnp.float32)
        + b_ref[...]
    )


def _matmul_bias(ctx_sum, dense_w, dense_b2d):
    B, D = ctx_sum.shape
    N = dense_w.shape[1]
    nblocks = pl.cdiv(N, _BN)
    return pl.pallas_call(
        _mm_body,
        grid=(nblocks,),
        in_specs=[
            pl.BlockSpec((B, D), lambda n: (0, 0)),
            pl.BlockSpec((D, _BN), lambda n: (0, n)),
            pl.BlockSpec((1, _BN), lambda n: (0, n)),
        ],
        out_specs=pl.BlockSpec((B, _BN), lambda n: (0, n)),
        out_shape=jax.ShapeDtypeStruct((B, N), jnp.float32),
    )(ctx_sum, dense_w, dense_b2d)


def kernel(context, emb_table, dense_w, dense_b):
    B, C = context.shape
    V, D = emb_table.shape
    N = dense_w.shape[1]
    info = plsc.get_sparse_core_info()
    NW = info.num_cores * info.num_subcores
    b_per_w = B // NW
    ctx_r = (context.astype(jnp.int32)
             .T.reshape(C, NW, b_per_w).transpose(1, 0, 2))
    table_p = jnp.pad(emb_table, ((0, 0), (0, _DP - D)))
    gather_sum = _make_gather_sum(V, D, B, C)
    ctx_sum = gather_sum(ctx_r, table_p)[:, :D]
    return _matmul_bias(ctx_sum, dense_w, dense_b.reshape(1, N))


# speedup vs baseline: 3.1567x; 3.1465x over previous
import functools

import jax
import jax.numpy as jnp
from jax import lax
from jax.experimental import pallas as pl
from jax.experimental.pallas import tpu as pltpu
from jax.experimental.pallas import tpu_sc as plsc

_DP = 128


def _make_gather_sum(V, D, B, C):
    info = plsc.get_sparse_core_info()
    NC, NS, L = info.num_cores, info.num_subcores, info.num_lanes
    NW = NC * NS
    assert B % NW == 0 and (B // NW) % 8 == 0
    assert D % L == 0
    b_per_w = B // NW
    slices_per_row = D // L
    mesh = plsc.VectorSubcoreMesh(core_axis_name="c", subcore_axis_name="s")

    @functools.partial(
        pl.kernel,
        mesh=mesh,
        out_type=jax.ShapeDtypeStruct((B, _DP), jnp.float32),
        scratch_types=[
            pltpu.VMEM((C, b_per_w), jnp.int32),
            pltpu.VMEM((b_per_w, _DP), jnp.float32),
            pltpu.VMEM((b_per_w, _DP), jnp.float32),
            pltpu.VMEM((b_per_w, _DP), jnp.float32),
            pltpu.SemaphoreType.DMA,
            pltpu.SemaphoreType.DMA,
            pltpu.SemaphoreType.DMA,
        ],
    )
    def gather_sum(ctx_hbm, table_hbm, out_hbm, idx_v, acc_v, buf0, buf1,
                   sem0, sem1, sem_a):
        wid = lax.axis_index("s") * NC + lax.axis_index("c")
        base = wid * b_per_w
        pltpu.sync_copy(ctx_hbm.at[wid], idx_v)

        bufs = (buf0, buf1)
        sems = (sem0, sem1)

        def add_into_acc(src):
            def row_body(i, _):
                for jj in range(slices_per_row):
                    sl = pl.ds(jj * L, L)
                    acc_v[i, sl] = acc_v[i, sl] + src[i, sl]
                return 0
            lax.fori_loop(0, b_per_w, row_body, 0)

        c0 = pltpu.async_copy(table_hbm.at[idx_v.at[0]], acc_v, sem_a)
        copies = [c0]
        for j in range(1, C):
            nb = bufs[(j - 1) % 2]
            copies.append(
                pltpu.async_copy(table_hbm.at[idx_v.at[j]], nb,
                                 sems[(j - 1) % 2]))
            copies[j - 1].wait()
            if j >= 2:
                add_into_acc(bufs[j % 2])
        copies[C - 1].wait()
        if C >= 2:
            add_into_acc(bufs[C % 2])
        pltpu.sync_copy(acc_v, out_hbm.at[pl.ds(base, b_per_w)])

    return gather_sum


_BN = 512


def _mm_body(w_ref, ctx_ref, b_ref, out_ref):
    out_ref[...] = (
        lax.dot_general(w_ref[...], ctx_ref[...],
                        dimension_numbers=(((0,), (1,)), ((), ())),
                        preferred_element_type=jnp.float32)
        + b_ref[...]
    )


def _matmul_bias_t(ctx_sum, dense_w, dense_b_col):
    B, D = ctx_sum.shape
    N = dense_w.shape[1]
    nblocks = pl.cdiv(N, _BN)
    return pl.pallas_call(
        _mm_body,
        grid=(nblocks,),
        in_specs=[
            pl.BlockSpec((D, _BN), lambda n: (0, n)),
            pl.BlockSpec((B, D), lambda n: (0, 0)),
            pl.BlockSpec((_BN, 1), lambda n: (n, 0)),
        ],
        out_specs=pl.BlockSpec((_BN, B), lambda n: (n, 0)),
        out_shape=jax.ShapeDtypeStruct((N, B), jnp.float32),
    )(dense_w, ctx_sum, dense_b_col)


def kernel(context, emb_table, dense_w, dense_b):
    B, C = context.shape
    V, D = emb_table.shape
    N = dense_w.shape[1]
    info = plsc.get_sparse_core_info()
    NW = info.num_cores * info.num_subcores
    b_per_w = B // NW
    ctx_r = (context.astype(jnp.int32)
             .T.reshape(C, NW, b_per_w).transpose(1, 0, 2))
    table_p = jnp.pad(emb_table, ((0, 0), (0, _DP - D)))
    gather_sum = _make_gather_sum(V, D, B, C)
    ctx_sum = gather_sum(ctx_r, table_p)[:, :D]
    logits_t = _matmul_bias_t(ctx_sum, dense_w, dense_b.reshape(N, 1))
    return logits_t.T
